# baseline (speedup 1.0000x reference)
import jax
import jax.numpy as jnp
from jax import lax
from jax.experimental import pallas as pl
from jax.experimental.pallas import tpu as pltpu
from jax.experimental.pallas import tpu_sc as plsc

_N = 20000
_NP = 20480
_NT = 16
_P = _NP // _NT
_R = _P // 16
_RF = _NP // 16
_MAX_OUT = 300
_OUT_ROWS = 304
_NEG = -1e30
_BIGI = 1 << 30


def _nms_sc(b0, b1, b2, b3, s_in, out_hbm,
            shr0, shr1, x1v, y1v, x2v, y2v, arv, alv, lcv, bsb, bib, redf,
            redi, outv):
    cid = lax.axis_index("c")
    sid = lax.axis_index("s")
    off = sid * _P
    li = lax.iota(jnp.int32, 16)
    zf = jnp.zeros((16,), jnp.float32)
    negv = jnp.full((16,), _NEG, jnp.float32)
    bigv = jnp.full((16,), _BIGI, jnp.int32)
    fifteen = jnp.full((16,), 15, jnp.int32)

    def maxsplat_f(x):
        redf[...] = plsc.cummax(x)
        return plsc.load_gather(redf, [fifteen])

    def minsplat_i(x):
        redi[...] = plsc.cummax(-x)
        return -plsc.load_gather(redi, [fifteen])

    pltpu.sync_copy(b0, x1v)
    pltpu.sync_copy(b1, y1v)
    pltpu.sync_copy(b2, x2v)
    pltpu.sync_copy(b3, y2v)
    pltpu.sync_copy(s_in.at[pl.ds(off, _P)], alv)

    def canon(r):
        d = pl.ds(r * 16, 16)
        a, b = x1v[d], x2v[d]
        lo, hi = jnp.minimum(a, b), jnp.maximum(a, b)
        x1v[d] = lo
        x2v[d] = hi
        p, q = y1v[d], y2v[d]
        lo2, hi2 = jnp.minimum(p, q), jnp.maximum(p, q)
        y1v[d] = lo2
        y2v[d] = hi2
        arv[d] = (hi - lo) * (hi2 - lo2)

    plsc.parallel_loop(0, _RF, unroll=8)(canon)

    def stage(bs, bi, wsh):
        bsb[...] = bs
        bib[...] = plsc.bitcast(bi, jnp.float32)
        pltpu.sync_copy(bsb, wsh.at[pl.ds(sid * 16, 16)])
        pltpu.sync_copy(bib, wsh.at[pl.ds(256 + sid * 16, 16)])

    def prescan(r, carry):
        bs, bi = carry
        d = pl.ds(r * 16, 16)
        a = alv[d]
        idx = off + r * 16 + li
        upd = a > bs
        return (jnp.where(upd, a, bs), jnp.where(upd, idx, bi))

    bs0, bi0 = plsc.parallel_loop(0, _R, unroll=4,
                                  carry=(negv, bigv))(prescan)
    stage(bs0, bi0, shr0)
    for i in range(_OUT_ROWS - _MAX_OUT):
        outv[pl.ds((_MAX_OUT + i) * 16, 16)] = zf
    plsc.subcore_barrier()

    def one_iter(t, rsh, wsh):
        pltpu.sync_copy(rsh, lcv)
        bs_rows = [lcv[pl.ds(k * 16, 16)] for k in range(_NT)]
        mv = bs_rows
        while len(mv) > 1:
            mv = [jnp.maximum(mv[2 * j], mv[2 * j + 1])
                  for j in range(len(mv) // 2)]
        m = maxsplat_f(mv[0])
        kv = [jnp.where(bs_rows[k] == m,
                        plsc.bitcast(lcv[pl.ds(256 + k * 16, 16)], jnp.int32),
                        bigv)
              for k in range(_NT)]
        while len(kv) > 1:
            kv = [jnp.minimum(kv[2 * j], kv[2 * j + 1])
                  for j in range(len(kv) // 2)]
        mi = minsplat_i(kv[0])
        valid = m[0] > jnp.float32(-5e29)
        gi = jnp.clip(mi, 0, _NP - 1)
        wx1 = plsc.load_gather(x1v, [gi])
        wy1 = plsc.load_gather(y1v, [gi])
        wx2 = plsc.load_gather(x2v, [gi])
        wy2 = plsc.load_gather(y2v, [gi])
        war = plsc.load_gather(arv, [gi])

        @pl.when((cid == 0) & (sid == 0))
        def _():
            vf = jnp.where(valid, jnp.float32(1.0), jnp.float32(0.0))
            row = jnp.where(li == 0, wx1, zf)
            row = jnp.where(li == 1, wy1, row)
            row = jnp.where(li == 2, wx2, row)
            row = jnp.where(li == 3, wy2, row)
            row = jnp.where(li == 4, m, row)
            outv[pl.ds(t * 16, 16)] = row * vf

        @pl.when(valid)
        def _():
            mlw = mi - jnp.full((16,), off, jnp.int32)
            inb = (mlw >= 0) & (mlw < _P) & (li == 0)
            plsc.store_scatter(alv, [jnp.clip(mlw, 0, _P - 1)], negv,
                               mask=inb)

            def supp(r, carry):
                bs, bi = carry
                d = pl.ds(r * 16, 16)
                df = pl.ds(off + r * 16, 16)
                a = alv[d]
                xx1, yy1 = x1v[df], y1v[df]
                xx2, yy2, ar = x2v[df], y2v[df], arv[df]
                iw = jnp.maximum(jnp.minimum(xx2, wx2)
                                 - jnp.maximum(xx1, wx1), 0.0)
                ih = jnp.maximum(jnp.minimum(yy2, wy2)
                                 - jnp.maximum(yy1, wy1), 0.0)
                inter = iw * ih
                iou = inter / (ar + war - inter + jnp.float32(1e-8))
                a2 = jnp.where(iou > jnp.float32(0.5), negv, a)
                alv[d] = a2
                idx = off + r * 16 + li
                upd = a2 > bs
                return (jnp.where(upd, a2, bs), jnp.where(upd, idx, bi))

            bs, bi = plsc.parallel_loop(0, _R, unroll=4,
                                        carry=(negv, bigv))(supp)
            stage(bs, bi, wsh)

        @pl.when(jnp.logical_not(valid))
        def _():
            stage(negv, bigv, wsh)

        plsc.subcore_barrier()

    def iter_pair(k, c):
        one_iter(2 * k, shr0, shr1)
        one_iter(2 * k + 1, shr1, shr0)
        return c

    lax.fori_loop(0, _MAX_OUT // 2, iter_pair, 0)

    @pl.when((cid == 0) & (sid == 0))
    def _():
        pltpu.sync_copy(outv, out_hbm)


@jax.jit
def _run(b0, b1, b2, b3, sp):
    mesh = plsc.VectorSubcoreMesh(core_axis_name="c", subcore_axis_name="s",
                                  num_cores=2, num_subcores=16)
    f = pl.kernel(
        _nms_sc,
        out_type=jax.ShapeDtypeStruct((_OUT_ROWS * 16,), jnp.float32),
        mesh=mesh,
        compiler_params=pltpu.CompilerParams(needs_layout_passes=False),
        scratch_types=[
            pltpu.VMEM_SHARED((512,), jnp.float32),
            pltpu.VMEM_SHARED((512,), jnp.float32),
            pltpu.VMEM((_NP,), jnp.float32),
            pltpu.VMEM((_NP,), jnp.float32),
            pltpu.VMEM((_NP,), jnp.float32),
            pltpu.VMEM((_NP,), jnp.float32),
            pltpu.VMEM((_NP,), jnp.float32),
            pltpu.VMEM((_P,), jnp.float32),
            pltpu.VMEM((512,), jnp.float32),
            pltpu.VMEM((16,), jnp.float32),
            pltpu.VMEM((16,), jnp.float32),
            pltpu.VMEM((16,), jnp.float32),
            pltpu.VMEM((16,), jnp.int32),
            pltpu.VMEM((_OUT_ROWS * 16,), jnp.float32),
        ],
    )
    return f(b0, b1, b2, b3, sp)


def kernel(boxes, scores):
    bT = jnp.zeros((4, _NP), jnp.float32).at[:, :_N].set(boxes.T)
    sp = jnp.full((_NP,), _NEG, jnp.float32).at[:_N].set(scores)
    out = _run(bT[0], bT[1], bT[2], bT[3], sp)
    return out.reshape(_OUT_ROWS, 16)[:_MAX_OUT, :5]


# speedup vs baseline: 1.0827x; 1.0827x over previous
import jax
import jax.numpy as jnp
from jax import lax
from jax.experimental import pallas as pl
from jax.experimental.pallas import tpu as pltpu
from jax.experimental.pallas import tpu_sc as plsc

_N = 20000
_NP = 20480
_NT = 16
_P = _NP // _NT
_R = _P // 16
_RF = _NP // 16
_MAX_OUT = 300
_OUT_ROWS = 304
_NEG = -1e30
_BIGI = 1 << 30


def _nms_sc(b0, b1, b2, b3, s_in, out_hbm,
            shr0, shr1, x1v, y1v, x2v, y2v, arv, alv, lcv, bsb, redf,
            redi, outv):
    cid = lax.axis_index("c")
    sid = lax.axis_index("s")
    off = sid * _P
    li = lax.iota(jnp.int32, 16)
    zf = jnp.zeros((16,), jnp.float32)
    negv = jnp.full((16,), _NEG, jnp.float32)
    bigv = jnp.full((16,), _BIGI, jnp.int32)
    fifteen = jnp.full((16,), 15, jnp.int32)

    def maxsplat_f(x):
        redf[...] = plsc.cummax(x)
        return plsc.load_gather(redf, [fifteen])

    def minsplat_i(x):
        redi[...] = plsc.cummax(-x)
        return -plsc.load_gather(redi, [fifteen])

    pltpu.sync_copy(b0, x1v)
    pltpu.sync_copy(b1, y1v)
    pltpu.sync_copy(b2, x2v)
    pltpu.sync_copy(b3, y2v)
    pltpu.sync_copy(s_in.at[pl.ds(off, _P)], alv)

    def canon(r):
        d = pl.ds(r * 16, 16)
        a, b = x1v[d], x2v[d]
        lo, hi = jnp.minimum(a, b), jnp.maximum(a, b)
        x1v[d] = lo
        x2v[d] = hi
        p, q = y1v[d], y2v[d]
        lo2, hi2 = jnp.minimum(p, q), jnp.maximum(p, q)
        y1v[d] = lo2
        y2v[d] = hi2
        arv[d] = (hi - lo) * (hi2 - lo2)

    plsc.parallel_loop(0, _RF, unroll=8)(canon)

    def stage(bs, bi, wsh):
        t1 = maxsplat_f(bs)
        k1 = minsplat_i(jnp.where(bs == t1, bi, bigv))
        row = jnp.where(li == 0, t1, zf)
        row = jnp.where(li == 1, plsc.bitcast(k1, jnp.float32), row)
        bsb[...] = row
        pltpu.sync_copy(bsb, wsh.at[pl.ds(sid * 16, 16)])

    def prescan(r, carry):
        bs, bi = carry
        d = pl.ds(r * 16, 16)
        a = alv[d]
        idx = off + r * 16 + li
        upd = a > bs
        return (jnp.where(upd, a, bs), jnp.where(upd, idx, bi))

    bs0, bi0 = plsc.parallel_loop(0, _R, unroll=4,
                                  carry=(negv, bigv))(prescan)
    stage(bs0, bi0, shr0)
    for i in range(_OUT_ROWS - _MAX_OUT):
        outv[pl.ds((_MAX_OUT + i) * 16, 16)] = zf
    plsc.subcore_barrier()

    def one_iter(t, rsh, wsh):
        pltpu.sync_copy(rsh, lcv)
        l16 = li * 16
        s1v = plsc.load_gather(lcv, [l16])
        i1v = plsc.bitcast(plsc.load_gather(lcv, [l16 + 1]), jnp.int32)
        m = maxsplat_f(s1v)
        mi = minsplat_i(jnp.where(s1v == m, i1v, bigv))
        valid = m[0] > jnp.float32(-5e29)
        gi = jnp.clip(mi, 0, _NP - 1)
        wx1 = plsc.load_gather(x1v, [gi])
        wy1 = plsc.load_gather(y1v, [gi])
        wx2 = plsc.load_gather(x2v, [gi])
        wy2 = plsc.load_gather(y2v, [gi])
        war = plsc.load_gather(arv, [gi])

        @pl.when((cid == 0) & (sid == 0))
        def _():
            vf = jnp.where(valid, jnp.float32(1.0), jnp.float32(0.0))
            row = jnp.where(li == 0, wx1, zf)
            row = jnp.where(li == 1, wy1, row)
            row = jnp.where(li == 2, wx2, row)
            row = jnp.where(li == 3, wy2, row)
            row = jnp.where(li == 4, m, row)
            outv[pl.ds(t * 16, 16)] = row * vf

        @pl.when(valid)
        def _():
            mlw = mi - jnp.full((16,), off, jnp.int32)
            inb = (mlw >= 0) & (mlw < _P) & (li == 0)
            plsc.store_scatter(alv, [jnp.clip(mlw, 0, _P - 1)], negv,
                               mask=inb)

            def supp(r, carry):
                bs, bi = carry
                d = pl.ds(r * 16, 16)
                df = pl.ds(off + r * 16, 16)
                a = alv[d]
                xx1, yy1 = x1v[df], y1v[df]
                xx2, yy2, ar = x2v[df], y2v[df], arv[df]
                iw = jnp.maximum(jnp.minimum(xx2, wx2)
                                 - jnp.maximum(xx1, wx1), 0.0)
                ih = jnp.maximum(jnp.minimum(yy2, wy2)
                                 - jnp.maximum(yy1, wy1), 0.0)
                inter = iw * ih
                iou = inter / (ar + war - inter + jnp.float32(1e-8))
                a2 = jnp.where(iou > jnp.float32(0.5), negv, a)
                alv[d] = a2
                idx = off + r * 16 + li
                upd = a2 > bs
                return (jnp.where(upd, a2, bs), jnp.where(upd, idx, bi))

            bs, bi = plsc.parallel_loop(0, _R, unroll=4,
                                        carry=(negv, bigv))(supp)
            stage(bs, bi, wsh)

        @pl.when(jnp.logical_not(valid))
        def _():
            stage(negv, bigv, wsh)

        plsc.subcore_barrier()

    def iter_pair(k, c):
        one_iter(2 * k, shr0, shr1)
        one_iter(2 * k + 1, shr1, shr0)
        return c

    lax.fori_loop(0, _MAX_OUT // 2, iter_pair, 0)

    @pl.when((cid == 0) & (sid == 0))
    def _():
        pltpu.sync_copy(outv, out_hbm)


@jax.jit
def _run(b0, b1, b2, b3, sp):
    mesh = plsc.VectorSubcoreMesh(core_axis_name="c", subcore_axis_name="s",
                                  num_cores=2, num_subcores=16)
    f = pl.kernel(
        _nms_sc,
        out_type=jax.ShapeDtypeStruct((_OUT_ROWS * 16,), jnp.float32),
        mesh=mesh,
        compiler_params=pltpu.CompilerParams(needs_layout_passes=False),
        scratch_types=[
            pltpu.VMEM_SHARED((256,), jnp.float32),
            pltpu.VMEM_SHARED((256,), jnp.float32),
            pltpu.VMEM((_NP,), jnp.float32),
            pltpu.VMEM((_NP,), jnp.float32),
            pltpu.VMEM((_NP,), jnp.float32),
            pltpu.VMEM((_NP,), jnp.float32),
            pltpu.VMEM((_NP,), jnp.float32),
            pltpu.VMEM((_P,), jnp.float32),
            pltpu.VMEM((256,), jnp.float32),
            pltpu.VMEM((16,), jnp.float32),
            pltpu.VMEM((16,), jnp.float32),
            pltpu.VMEM((16,), jnp.int32),
            pltpu.VMEM((_OUT_ROWS * 16,), jnp.float32),
        ],
    )
    return f(b0, b1, b2, b3, sp)


def kernel(boxes, scores):
    bT = jnp.zeros((4, _NP), jnp.float32).at[:, :_N].set(boxes.T)
    sp = jnp.full((_NP,), _NEG, jnp.float32).at[:_N].set(scores)
    out = _run(bT[0], bT[1], bT[2], bT[3], sp)
    return out.reshape(_OUT_ROWS, 16)[:_MAX_OUT, :5]


# speedup vs baseline: 1.2257x; 1.1321x over previous
import jax
import jax.numpy as jnp
from jax import lax
from jax.experimental import pallas as pl
from jax.experimental.pallas import tpu as pltpu
from jax.experimental.pallas import tpu_sc as plsc

_N = 20000
_NP = 20480
_NT = 16
_P = _NP // _NT
_R = _P // 16
_RF = _NP // 16
_MAX_OUT = 300
_OUT_ROWS = 304
_NEG = -1e30
_BIGI = 1 << 30


def _nms_sc(b0, b1, b2, b3, s_in, out_hbm,
            shr0, shr1, x1v, y1v, x2v, y2v, arv, alv, lcv, bsb, redf,
            redi, outv):
    cid = lax.axis_index("c")
    sid = lax.axis_index("s")
    off = sid * _P
    li = lax.iota(jnp.int32, 16)
    zi = jnp.zeros((16,), jnp.int32)
    zf = jnp.zeros((16,), jnp.float32)
    negv = jnp.full((16,), _NEG, jnp.float32)
    bigv = jnp.full((16,), _BIGI, jnp.int32)
    fifteen = jnp.full((16,), 15, jnp.int32)

    def maxsplat_f(x):
        redf[...] = plsc.cummax(x)
        return plsc.load_gather(redf, [fifteen])

    def minsplat_i(x):
        redi[...] = plsc.cummax(-x)
        return -plsc.load_gather(redi, [fifteen])

    pltpu.sync_copy(b0, x1v)
    pltpu.sync_copy(b1, y1v)
    pltpu.sync_copy(b2, x2v)
    pltpu.sync_copy(b3, y2v)
    pltpu.sync_copy(s_in.at[pl.ds(off, _P)], alv)

    def canon(r):
        d = pl.ds(r * 16, 16)
        a, b = x1v[d], x2v[d]
        lo, hi = jnp.minimum(a, b), jnp.maximum(a, b)
        x1v[d] = lo
        x2v[d] = hi
        p, q = y1v[d], y2v[d]
        lo2, hi2 = jnp.minimum(p, q), jnp.maximum(p, q)
        y1v[d] = lo2
        y2v[d] = hi2
        arv[d] = (hi - lo) * (hi2 - lo2)

    plsc.parallel_loop(0, _RF, unroll=8)(canon)

    def stage(b1s, i1s, b2s, i2s, wsh):
        t1 = maxsplat_f(b1s)
        k1 = minsplat_i(jnp.where(b1s == t1, i1s, bigv))
        selm = (b1s == t1) & (i1s == k1)
        xx = jnp.where(selm, b2s, b1s)
        ixx = jnp.where(selm, i2s, i1s)
        t2 = maxsplat_f(xx)
        k2 = minsplat_i(jnp.where(xx == t2, ixx, bigv))
        row = jnp.where(li == 0, t1, zf)
        row = jnp.where(li == 1, plsc.bitcast(k1, jnp.float32), row)
        row = jnp.where(li == 2, t2, row)
        row = jnp.where(li == 3, plsc.bitcast(k2, jnp.float32), row)
        bsb[...] = row
        pltpu.sync_copy(bsb, wsh.at[pl.ds(sid * 16, 16)])

    def top2upd(carry, a2, idx):
        c1, j1, c2, j2 = carry
        u1 = a2 > c1
        u2 = a2 > c2
        nb2 = jnp.where(u1, c1, jnp.where(u2, a2, c2))
        ni2 = jnp.where(u1, j1, jnp.where(u2, idx, j2))
        return (jnp.where(u1, a2, c1), jnp.where(u1, idx, j1), nb2, ni2)

    def prescan(r, carry):
        d = pl.ds(r * 16, 16)
        a = alv[d]
        idx = off + r * 16 + li
        return top2upd(carry, a, idx)

    c0 = plsc.parallel_loop(0, _R, unroll=4,
                            carry=(negv, bigv, negv, bigv))(prescan)
    stage(*c0, shr0)

    def zout(r):
        outv[pl.ds(r * 16, 16)] = zf

    plsc.parallel_loop(0, _OUT_ROWS, unroll=4)(zout)
    plsc.subcore_barrier()

    def iou_vs(ax1, ay1, ax2, ay2, aar, ux1, uy1, ux2, uy2, uar):
        iw = jnp.maximum(jnp.minimum(ax2, ux2) - jnp.maximum(ax1, ux1), 0.0)
        ih = jnp.maximum(jnp.minimum(ay2, uy2) - jnp.maximum(ay1, uy1), 0.0)
        inter = iw * ih
        return inter / (aar + uar - inter + jnp.float32(1e-8))

    def gath5(g):
        return (plsc.load_gather(x1v, [g]), plsc.load_gather(y1v, [g]),
                plsc.load_gather(x2v, [g]), plsc.load_gather(y2v, [g]),
                plsc.load_gather(arv, [g]))

    def one_iter(rsh, wsh, cnt):
        pltpu.sync_copy(rsh, lcv)
        l16 = li * 16
        s1v = plsc.load_gather(lcv, [l16])
        i1v = plsc.bitcast(plsc.load_gather(lcv, [l16 + 1]), jnp.int32)
        s2v = plsc.load_gather(lcv, [l16 + 2])
        i2v = plsc.bitcast(plsc.load_gather(lcv, [l16 + 3]), jnp.int32)
        m = maxsplat_f(jnp.maximum(s1v, s2v))
        mi = minsplat_i(jnp.minimum(jnp.where(s1v == m, i1v, bigv),
                                    jnp.where(s2v == m, i2v, bigv)))
        valid = m[0] > jnp.float32(-5e29)
        w1 = gath5(jnp.clip(mi, 0, _NP - 1))
        e1 = gath5(jnp.clip(i1v, 0, _NP - 1))
        e2 = gath5(jnp.clip(i2v, 0, _NP - 1))
        al1 = ((s1v > jnp.float32(-5e29)) & (i1v != mi)
               & jnp.logical_not(iou_vs(*e1, *w1) > jnp.float32(0.5)))
        al2 = ((s2v > jnp.float32(-5e29)) & (i2v != mi)
               & jnp.logical_not(iou_vs(*e2, *w1) > jnp.float32(0.5)))
        has = s1v > jnp.float32(-5e29)
        okv = jnp.where(jnp.logical_not(has) | al1 | al2,
                        jnp.full((16,), 1, jnp.int32), zi)
        allok = minsplat_i(okv)
        c1 = jnp.where(al1, s1v, negv)
        c2 = jnp.where(al2, s2v, negv)
        m2 = maxsplat_f(jnp.maximum(c1, c2))
        mi2 = minsplat_i(jnp.minimum(jnp.where(al1 & (c1 == m2), i1v, bigv),
                                     jnp.where(al2 & (c2 == m2), i2v, bigv)))
        w2ok = valid & (allok[0] > 0) & (m2[0] > jnp.float32(-5e29))
        w2f = (zi + jnp.where(w2ok, jnp.int32(1), jnp.int32(0))) > 0
        v = gath5(jnp.clip(mi2, 0, _NP - 1))
        nop = jnp.full((16,), -5.0, jnp.float32)
        w2 = (jnp.where(w2f, v[0], nop), jnp.where(w2f, v[1], nop),
              jnp.where(w2f, v[2], nop), jnp.where(w2f, v[3], nop), v[4])

        @pl.when((cid == 0) & (sid == 0) & valid & (cnt < _MAX_OUT))
        def _():
            row = jnp.where(li == 0, w1[0], zf)
            row = jnp.where(li == 1, w1[1], row)
            row = jnp.where(li == 2, w1[2], row)
            row = jnp.where(li == 3, w1[3], row)
            row = jnp.where(li == 4, m, row)
            outv[pl.ds(cnt * 16, 16)] = row

        @pl.when((cid == 0) & (sid == 0) & w2ok & (cnt + 1 < _MAX_OUT))
        def _():
            row = jnp.where(li == 0, w2[0], zf)
            row = jnp.where(li == 1, w2[1], row)
            row = jnp.where(li == 2, w2[2], row)
            row = jnp.where(li == 3, w2[3], row)
            row = jnp.where(li == 4, m2, row)
            outv[pl.ds((cnt + 1) * 16, 16)] = row

        @pl.when(valid)
        def _():
            offv = jnp.full((16,), off, jnp.int32)
            mlw = mi - offv
            inb = (mlw >= 0) & (mlw < _P) & (li == 0)
            plsc.store_scatter(alv, [jnp.clip(mlw, 0, _P - 1)], negv,
                               mask=inb)
            mlw2 = mi2 - offv
            inb2 = (mlw2 >= 0) & (mlw2 < _P) & (li == 0) & w2f
            plsc.store_scatter(alv, [jnp.clip(mlw2, 0, _P - 1)], negv,
                               mask=inb2)

            def supp(r, carry):
                d = pl.ds(r * 16, 16)
                df = pl.ds(off + r * 16, 16)
                a = alv[d]
                e = (x1v[df], y1v[df], x2v[df], y2v[df], arv[df])
                kill = ((iou_vs(*e, *w1) > jnp.float32(0.5))
                        | (iou_vs(*e, *w2) > jnp.float32(0.5)))
                a2 = jnp.where(kill, negv, a)
                alv[d] = a2
                idx = off + r * 16 + li
                return top2upd(carry, a2, idx)

            c = plsc.parallel_loop(0, _R, unroll=4,
                                   carry=(negv, bigv, negv, bigv))(supp)
            stage(*c, wsh)

        @pl.when(jnp.logical_not(valid))
        def _():
            stage(negv, bigv, negv, bigv, wsh)

        plsc.subcore_barrier()
        ncnt = cnt + jnp.where(valid, jnp.int32(1), jnp.int32(0)) \
            + jnp.where(w2ok, jnp.int32(1), jnp.int32(0))
        return ncnt, valid

    def pair_body(carry):
        k, cnt, _ = carry
        cnt, _ = one_iter(shr0, shr1, cnt)
        cnt, cont = one_iter(shr1, shr0, cnt)
        return (k + jnp.int32(1), cnt, cont)

    def pair_cond(carry):
        k, cnt, cont = carry
        return (k < _MAX_OUT // 2) & cont & (cnt < _MAX_OUT)

    lax.while_loop(pair_cond, pair_body,
                   (jnp.int32(0), jnp.int32(0), jnp.bool_(True)))

    @pl.when((cid == 0) & (sid == 0))
    def _():
        pltpu.sync_copy(outv, out_hbm)


@jax.jit
def _run(b0, b1, b2, b3, sp):
    mesh = plsc.VectorSubcoreMesh(core_axis_name="c", subcore_axis_name="s",
                                  num_cores=2, num_subcores=16)
    f = pl.kernel(
        _nms_sc,
        out_type=jax.ShapeDtypeStruct((_OUT_ROWS * 16,), jnp.float32),
        mesh=mesh,
        compiler_params=pltpu.CompilerParams(needs_layout_passes=False),
        scratch_types=[
            pltpu.VMEM_SHARED((256,), jnp.float32),
            pltpu.VMEM_SHARED((256,), jnp.float32),
            pltpu.VMEM((_NP,), jnp.float32),
            pltpu.VMEM((_NP,), jnp.float32),
            pltpu.VMEM((_NP,), jnp.float32),
            pltpu.VMEM((_NP,), jnp.float32),
            pltpu.VMEM((_NP,), jnp.float32),
            pltpu.VMEM((_P,), jnp.float32),
            pltpu.VMEM((256,), jnp.float32),
            pltpu.VMEM((16,), jnp.float32),
            pltpu.VMEM((16,), jnp.float32),
            pltpu.VMEM((16,), jnp.int32),
            pltpu.VMEM((_OUT_ROWS * 16,), jnp.float32),
        ],
    )
    return f(b0, b1, b2, b3, sp)


def kernel(boxes, scores):
    bT = jnp.zeros((4, _NP), jnp.float32).at[:, :_N].set(boxes.T)
    sp = jnp.full((_NP,), _NEG, jnp.float32).at[:_N].set(scores)
    out = _run(bT[0], bT[1], bT[2], bT[3], sp)
    return out.reshape(_OUT_ROWS, 16)[:_MAX_OUT, :5]
